# baseline (speedup 1.0000x reference)
import functools

import jax
import jax.numpy as jnp
from jax import lax
from jax.experimental import pallas as pl
from jax.experimental.pallas import tpu as pltpu
from jax.experimental.pallas import tpu_sc as plsc

DIM = 32
NC, NS = 2, 16
NW = NC * NS
BATCH_PER_W = 512
STEP_BATCH = 16


def _tc_project(table, Wt):
    m = table.shape[0]
    bm = 4000
    grid = m // bm

    def body(x_ref, w_ref, o_ref):
        o_ref[...] = jnp.dot(x_ref[...], w_ref[...],
                             preferred_element_type=jnp.float32)

    return pl.pallas_call(
        body,
        grid=(grid,),
        in_specs=[pl.BlockSpec((bm, DIM), lambda i: (i, 0)),
                  pl.BlockSpec((DIM, DIM), lambda i: (0, 0))],
        out_specs=pl.BlockSpec((bm, DIM), lambda i: (i, 0)),
        out_shape=jax.ShapeDtypeStruct((m, DIM), jnp.float32),
    )(table, Wt)


def _sc_gather(ptable, indexes):
    bsz, f = indexes.shape
    mesh = plsc.VectorSubcoreMesh(core_axis_name="c", subcore_axis_name="s")
    steps = BATCH_PER_W // STEP_BATCH

    @functools.partial(
        pl.kernel,
        mesh=mesh,
        compiler_params=pltpu.CompilerParams(use_tc_tiling_on_sc=False),
        out_type=jax.ShapeDtypeStruct((bsz, f, DIM), jnp.float32),
        scratch_types=[
            pltpu.VMEM((BATCH_PER_W, f), jnp.int32),
            pltpu.VMEM((STEP_BATCH, f, DIM), jnp.float32),
            pltpu.SemaphoreType.DMA,
        ],
    )
    def k(table_hbm, idx_hbm, out_hbm, idx_v, rows_v, sem):
        wid = lax.axis_index("s") * NC + lax.axis_index("c")
        batch0 = wid * BATCH_PER_W
        pltpu.sync_copy(idx_hbm.at[pl.ds(batch0, BATCH_PER_W)], idx_v)

        def step(s, carry):
            copies = []
            for t in range(STEP_BATCH):
                copies.append(pltpu.async_copy(
                    table_hbm.at[idx_v.at[s * STEP_BATCH + t]],
                    rows_v.at[t],
                    sem,
                ))
            for c in copies:
                c.wait()
            pltpu.sync_copy(
                rows_v,
                out_hbm.at[pl.ds(batch0 + s * STEP_BATCH, STEP_BATCH)])
            return carry

        lax.fori_loop(0, steps, step, 0)

    return k(ptable, indexes)


def kernel(indexes, table, W):
    idx = indexes.astype(jnp.int32)
    ptable = _tc_project(table, W.T)
    return _sc_gather(ptable, idx)


# speedup vs baseline: 1.5471x; 1.5471x over previous
import functools

import jax
import jax.numpy as jnp
from jax import lax
from jax.experimental import pallas as pl
from jax.experimental.pallas import tpu as pltpu
from jax.experimental.pallas import tpu_sc as plsc

DIM = 32
NC, NS = 2, 16
NW = NC * NS
BATCH_PER_W = 512
STEP_BATCH = 16


def _sc_gather(table, indexes):
    bsz, f = indexes.shape
    mesh = plsc.VectorSubcoreMesh(core_axis_name="c", subcore_axis_name="s")
    steps = BATCH_PER_W // STEP_BATCH

    @functools.partial(
        pl.kernel,
        mesh=mesh,
        compiler_params=pltpu.CompilerParams(use_tc_tiling_on_sc=False),
        out_type=jax.ShapeDtypeStruct((bsz, f, DIM), jnp.float32),
        scratch_types=[
            pltpu.VMEM((BATCH_PER_W, f), jnp.int32),
            pltpu.VMEM((STEP_BATCH, f, DIM), jnp.float32),
            pltpu.SemaphoreType.DMA,
        ],
    )
    def k(table_hbm, idx_hbm, out_hbm, idx_v, rows_v, sem):
        wid = lax.axis_index("s") * NC + lax.axis_index("c")
        batch0 = wid * BATCH_PER_W
        pltpu.sync_copy(idx_hbm.at[pl.ds(batch0, BATCH_PER_W)], idx_v)

        def step(s, carry):
            copies = []
            for t in range(STEP_BATCH):
                copies.append(pltpu.async_copy(
                    table_hbm.at[idx_v.at[s * STEP_BATCH + t]],
                    rows_v.at[t],
                    sem,
                ))
            for c in copies:
                c.wait()
            pltpu.sync_copy(
                rows_v,
                out_hbm.at[pl.ds(batch0 + s * STEP_BATCH, STEP_BATCH)])
            return carry

        lax.fori_loop(0, steps, step, 0)

    return k(table, indexes)


def _tc_project(emb4, wbig):
    m = emb4.shape[0]
    bm = 4096
    grid = m // bm

    def body(x_ref, w_ref, o_ref):
        o_ref[...] = jnp.dot(x_ref[...], w_ref[...],
                             preferred_element_type=jnp.float32)

    return pl.pallas_call(
        body,
        grid=(grid,),
        in_specs=[pl.BlockSpec((bm, 128), lambda i: (i, 0)),
                  pl.BlockSpec((128, 128), lambda i: (0, 0))],
        out_specs=pl.BlockSpec((bm, 128), lambda i: (i, 0)),
        out_shape=jax.ShapeDtypeStruct((m, 128), jnp.float32),
    )(emb4, wbig)


def kernel(indexes, table, W):
    b, f = indexes.shape
    n = b * f
    idx = indexes.astype(jnp.int32)
    emb = _sc_gather(table, idx)
    wbig = jnp.kron(jnp.eye(4, dtype=jnp.float32), W.T)
    out4 = _tc_project(emb.reshape(n // 4, 128), wbig)
    return out4.reshape(b, f, DIM)


# speedup vs baseline: 1.9348x; 1.2506x over previous
import functools

import jax
import jax.numpy as jnp
from jax import lax
from jax.experimental import pallas as pl
from jax.experimental.pallas import tpu as pltpu
from jax.experimental.pallas import tpu_sc as plsc

DIM = 32
NC, NS = 2, 16
NW = NC * NS
BATCH_PER_W = 512
STEP_BATCH = 16


def _tc_project_wide(tableT, w128):
    m = tableT.shape[1]
    bn = 8192
    grid = pl.cdiv(m, bn)

    def body(x_ref, w_ref, o_ref):
        o_ref[...] = lax.dot_general(
            x_ref[...], w_ref[...], (((0,), (0,)), ((), ())),
            preferred_element_type=jnp.float32)

    return pl.pallas_call(
        body,
        grid=(grid,),
        in_specs=[pl.BlockSpec((DIM, bn), lambda i: (0, i)),
                  pl.BlockSpec((DIM, 128), lambda i: (0, 0))],
        out_specs=pl.BlockSpec((bn, 128), lambda i: (i, 0)),
        out_shape=jax.ShapeDtypeStruct((m, 128), jnp.float32),
    )(tableT, w128)


def _sc_gather(ptable, indexes):
    bsz, f = indexes.shape
    mesh = plsc.VectorSubcoreMesh(core_axis_name="c", subcore_axis_name="s")
    steps = BATCH_PER_W // STEP_BATCH

    @functools.partial(
        pl.kernel,
        mesh=mesh,
        compiler_params=pltpu.CompilerParams(use_tc_tiling_on_sc=False),
        out_type=jax.ShapeDtypeStruct((bsz, f, DIM), jnp.float32),
        scratch_types=[
            pltpu.VMEM((BATCH_PER_W, f), jnp.int32),
            pltpu.VMEM((STEP_BATCH, f, 128), jnp.float32),
            pltpu.SemaphoreType.DMA,
        ],
    )
    def k(table_hbm, idx_hbm, out_hbm, idx_v, rows_v, sem):
        wid = lax.axis_index("s") * NC + lax.axis_index("c")
        batch0 = wid * BATCH_PER_W
        pltpu.sync_copy(idx_hbm.at[pl.ds(batch0, BATCH_PER_W)], idx_v)

        def step(s, carry):
            copies = []
            for t in range(STEP_BATCH):
                copies.append(pltpu.async_copy(
                    table_hbm.at[idx_v.at[s * STEP_BATCH + t]],
                    rows_v.at[t],
                    sem,
                ))
            for c in copies:
                c.wait()
            pltpu.sync_copy(
                rows_v.at[:, :, pl.ds(0, DIM)],
                out_hbm.at[pl.ds(batch0 + s * STEP_BATCH, STEP_BATCH)])
            return carry

        lax.fori_loop(0, steps, step, 0)

    return k(ptable, indexes)


def kernel(indexes, table, W):
    idx = indexes.astype(jnp.int32)
    w128 = jnp.pad(W.T, ((0, 0), (0, 96)))
    ptable = _tc_project_wide(table.T, w128)
    return _sc_gather(ptable, idx)
